# baseline (speedup 1.0000x reference)
import jax
import jax.numpy as jnp
from jax import lax
from jax.experimental import pallas as pl
from jax.experimental.pallas import tpu as pltpu
from jax.experimental.pallas import tpu_sc as plsc

B, L, D, K = 16, 4096, 1024, 32
CH = 128
CAP = 256
NSUB = 8
NTASK = 4
NLANE = 16
DUMP = 8 * 256 * 16

NEG = float("-inf")


def _bc_i32(s):
    return jnp.full((NLANE,), s, dtype=jnp.int32)


def _bc_f32(s):
    return jnp.full((NLANE,), s, dtype=jnp.float32)


def _ce_desc(a, b):
    return jnp.maximum(a, b), jnp.minimum(a, b)


def _bitonic_sort_desc(a):
    n = len(a)
    k = 2
    while k <= n:
        j = k // 2
        while j >= 1:
            for i in range(n):
                l = i ^ j
                if l > i:
                    hi, lo = _ce_desc(a[i], a[l])
                    if (i & k) == 0:
                        a[i], a[l] = hi, lo
                    else:
                        a[i], a[l] = lo, hi
            j //= 2
        k *= 2
    return a


def _bitonic_cleanup_desc(a):
    n = len(a)
    j = n // 2
    while j >= 1:
        for i in range(n):
            l = i ^ j
            if l > i:
                a[i], a[l] = _ce_desc(a[i], a[l])
        j //= 2
    return a


def _smax_i32(v, lanes):
    for d in (8, 4, 2, 1):
        perm = jnp.bitwise_xor(lanes, _bc_i32(d))
        v = jnp.maximum(v, v.at[perm].get(mode="promise_in_bounds"))
    return v[0]


def _tec_body(feat, len_hbm, out_hbm, len_v, idx0, idx1, idx2, chk0, chk1,
              chk2, cand, run, cnt, tbuf, outst, sem0, sem1, sem2):
    wid = lax.axis_index("s") * 2 + lax.axis_index("c")
    pltpu.sync_copy(len_hbm, len_v)
    lanes = lax.iota(jnp.int32, 16)

    def task_body(ti, _):
        task = ti * 32 + wid
        b = task // NSUB
        g0 = (task % NSUB) * NSUB
        lb = len_v[...].at[_bc_i32(b)].get(mode="promise_in_bounds")[0]

        def reset_run(i, c):
            run[i] = _bc_f32(NEG)
            return c
        lax.fori_loop(0, NSUB * K, reset_run, 0)

        def reset_state(i, c):
            cnt[i] = _bc_i32(0)
            tbuf[i] = _bc_f32(NEG)
            return c
        lax.fori_loop(0, NSUB, reset_state, 0)

        def flush_all(thresh):
            def per_sub(sub, c0):
                c = cnt[sub]
                maxc = _smax_i32(c, lanes)

                @pl.when(maxc > thresh * 16)
                def _():
                    nblk = (maxc + K * 16 - 1) // (K * 16)

                    def blk_body(blk, c1):
                        base = (sub * CAP + blk * K) * 16
                        s = []
                        for i in range(K):
                            x = cand[pl.ds(base + i * 16, 16)]
                            valid = _bc_i32((blk * K + i) * 16) < c
                            s.append(jnp.where(valid, x, _bc_f32(NEG)))
                        s = _bitonic_sort_desc(s)
                        rbase = sub * K
                        m = [jnp.maximum(run[rbase + i], s[K - 1 - i])
                             for i in range(K)]
                        m = _bitonic_cleanup_desc(m)
                        for i in range(K):
                            run[rbase + i] = m[i]
                        return c1
                    lax.fori_loop(0, nblk, blk_body, 0)
                    cnt[sub] = _bc_i32(0)
                    tbuf[sub] = run[sub * K + K - 1]
                return c0
            lax.fori_loop(0, NSUB, per_sub, 0)

        nch = (lb + CH - 1) // CH

        goff = task % NSUB

        def issue(cidx, ibuf, cbuf, sem):
            l0 = cidx * CH

            def idx_body(i, c2):
                rowid = _bc_i32(b * L + l0 + i * NLANE) + lanes
                ibuf[pl.ds(i * NLANE, NLANE)] = rowid * NSUB + _bc_i32(goff)
                return c2
            lax.fori_loop(0, CH // NLANE, idx_body, 0)
            pltpu.async_copy(feat.at[ibuf], cbuf, sem)

        def process(cidx, ibuf, cbuf, sem):
            pltpu.make_async_copy(feat.at[ibuf], cbuf, sem).wait()
            l0 = cidx * CH
            rows_here = jnp.minimum(CH, lb - l0)

            def sub_body(sub, c1):
                t = tbuf[sub]
                basel = _bc_i32(sub * CAP * 16) + lanes
                dumpl = _bc_i32(DUMP) + lanes
                sixteen = _bc_i32(16)
                zero = _bc_i32(0)

                def oct_body(q, cv):
                    r = q * 8
                    xs = [cbuf[r + j, pl.ds(sub * NLANE, NLANE)]
                          for j in range(8)]
                    ms = [x >= t for x in xs]
                    inc = [jnp.where(m, sixteen, zero) for m in ms]
                    offs = [cv]
                    for j in range(7):
                        offs.append(offs[-1] + inc[j])
                    for j in range(8):
                        idx = jnp.where(ms[j], basel + offs[j], dumpl)
                        plsc.store_scatter(cand, [idx], xs[j])
                    return offs[7] + inc[7]
                nq = rows_here // 8
                cv = lax.fori_loop(0, nq, oct_body, cnt[sub])

                def row_body(r, cv2):
                    x = cbuf[r, pl.ds(sub * NLANE, NLANE)]
                    msk = x >= t
                    idx = jnp.where(msk, basel + cv2, dumpl)
                    plsc.store_scatter(cand, [idx], x)
                    return cv2 + jnp.where(msk, sixteen, zero)
                cnt[sub] = lax.fori_loop(nq * 8, rows_here, row_body, cv)
                return c1
            lax.fori_loop(0, NSUB, sub_body, 0)
            flush_all(K)

        bufs = ((idx0, chk0, sem0), (idx1, chk1, sem1), (idx2, chk2, sem2))

        @pl.when(nch > 0)
        def _():
            for j in range(3):
                @pl.when(j < nch)
                def _(j=j):
                    issue(j, *bufs[j])

            def ring_body(p, c0):
                for j in range(3):
                    c = p * 3 + j

                    @pl.when(c < nch)
                    def _(c=c, j=j):
                        process(c, *bufs[j])

                    @pl.when(c + 3 < nch)
                    def _(c=c, j=j):
                        issue(c + 3, *bufs[j])
                return c0
            lax.fori_loop(0, (nch + 2) // 3, ring_body, 0)
        flush_all(0)

        ki = jnp.minimum(K, lb)
        kif = _bc_f32(ki.astype(jnp.float32))

        def out_body(sub, c0):
            acc = _bc_f32(0.0)
            for i in range(K):
                sel = _bc_i32(i) < _bc_i32(ki)
                acc = acc + jnp.where(sel, run[sub * K + i], _bc_f32(0.0))
            outst[pl.ds(sub * NLANE, NLANE)] = acc / kif
            return c0
        lax.fori_loop(0, NSUB, out_body, 0)
        pltpu.sync_copy(outst, out_hbm.at[pl.ds(b * D + (task % NSUB) * 128, 128)])
        return 0

    lax.fori_loop(0, NTASK, task_body, 0)


_sc_call = pl.kernel(
    _tec_body,
    out_type=jax.ShapeDtypeStruct((B * D,), jnp.float32),
    mesh=plsc.VectorSubcoreMesh(core_axis_name="c", subcore_axis_name="s"),
    compiler_params=pltpu.CompilerParams(needs_layout_passes=False,
                                        skip_device_barrier=True),
    scratch_types=[
        pltpu.VMEM((16,), jnp.int32),
        pltpu.VMEM((CH,), jnp.int32),
        pltpu.VMEM((CH,), jnp.int32),
        pltpu.VMEM((CH,), jnp.int32),
        pltpu.VMEM((CH, NSUB * 16), jnp.float32),
        pltpu.VMEM((CH, NSUB * 16), jnp.float32),
        pltpu.VMEM((CH, NSUB * 16), jnp.float32),
        pltpu.VMEM((NSUB * CAP * 16 + 16,), jnp.float32),
        pltpu.VMEM((NSUB * K, 16), jnp.float32),
        pltpu.VMEM((NSUB, 16), jnp.int32),
        pltpu.VMEM((NSUB, 16), jnp.float32),
        pltpu.VMEM((128,), jnp.float32),
        pltpu.SemaphoreType.DMA,
        pltpu.SemaphoreType.DMA,
        pltpu.SemaphoreType.DMA,
    ],
)


@jax.jit
def kernel(features, lengths):
    feat2 = features.reshape(B * L * NSUB, 128)
    return _sc_call(feat2, lengths).reshape(B, D)


# speedup vs baseline: 1.0360x; 1.0360x over previous
import jax
import jax.numpy as jnp
from jax import lax
from jax.experimental import pallas as pl
from jax.experimental.pallas import tpu as pltpu
from jax.experimental.pallas import tpu_sc as plsc

B, L, D, K = 16, 4096, 1024, 32
CH = 128
CAP = 384
NSUB = 8
NTASK = 4
NLANE = 16
DUMP = 8 * 384 * 16

NEG = float("-inf")


def _bc_i32(s):
    return jnp.full((NLANE,), s, dtype=jnp.int32)


def _bc_f32(s):
    return jnp.full((NLANE,), s, dtype=jnp.float32)


def _ce_desc(a, b):
    return jnp.maximum(a, b), jnp.minimum(a, b)


def _bitonic_sort_desc(a):
    n = len(a)
    k = 2
    while k <= n:
        j = k // 2
        while j >= 1:
            for i in range(n):
                l = i ^ j
                if l > i:
                    hi, lo = _ce_desc(a[i], a[l])
                    if (i & k) == 0:
                        a[i], a[l] = hi, lo
                    else:
                        a[i], a[l] = lo, hi
            j //= 2
        k *= 2
    return a


def _bitonic_cleanup_desc(a):
    n = len(a)
    j = n // 2
    while j >= 1:
        for i in range(n):
            l = i ^ j
            if l > i:
                a[i], a[l] = _ce_desc(a[i], a[l])
        j //= 2
    return a


def _smax_i32(v, lanes):
    for d in (8, 4, 2, 1):
        perm = jnp.bitwise_xor(lanes, _bc_i32(d))
        v = jnp.maximum(v, v.at[perm].get(mode="promise_in_bounds"))
    return v[0]


def _tec_body(feat, len_hbm, out_hbm, len_v, idx0, idx1, chk0, chk1, cand, run,
              cnt, tbuf, outst, sem0, sem1):
    wid = lax.axis_index("s") * 2 + lax.axis_index("c")
    pltpu.sync_copy(len_hbm, len_v)
    lanes = lax.iota(jnp.int32, 16)

    def task_body(ti, _):
        task = ti * 32 + wid
        b = task // NSUB
        g0 = (task % NSUB) * NSUB
        lb = len_v[...].at[_bc_i32(b)].get(mode="promise_in_bounds")[0]

        def reset_run(i, c):
            run[i] = _bc_f32(NEG)
            return c
        lax.fori_loop(0, NSUB * K, reset_run, 0)

        def reset_state(i, c):
            cnt[i] = _bc_i32(0)
            tbuf[i] = _bc_f32(NEG)
            return c
        lax.fori_loop(0, NSUB, reset_state, 0)

        def flush_all(thresh):
            def per_sub(sub, c0):
                c = cnt[sub]
                maxc = _smax_i32(c, lanes)

                @pl.when(maxc > thresh * 16)
                def _():
                    nblk = (maxc + K * 16 - 1) // (K * 16)

                    def blk_body(blk, c1):
                        base = (sub * CAP + blk * K) * 16
                        s = []
                        for i in range(K):
                            x = cand[pl.ds(base + i * 16, 16)]
                            valid = _bc_i32((blk * K + i) * 16) < c
                            s.append(jnp.where(valid, x, _bc_f32(NEG)))
                        s = _bitonic_sort_desc(s)
                        rbase = sub * K
                        m = [jnp.maximum(run[rbase + i], s[K - 1 - i])
                             for i in range(K)]
                        m = _bitonic_cleanup_desc(m)
                        for i in range(K):
                            run[rbase + i] = m[i]
                        return c1
                    lax.fori_loop(0, nblk, blk_body, 0)
                    cnt[sub] = _bc_i32(0)
                    tbuf[sub] = run[sub * K + K - 1]
                return c0
            lax.fori_loop(0, NSUB, per_sub, 0)

        nch = (lb + CH - 1) // CH

        goff = task % NSUB

        def issue(cidx, ibuf, cbuf, sem):
            l0 = cidx * CH

            def idx_body(i, c2):
                rowid = _bc_i32(b * L + l0 + i * NLANE) + lanes
                ibuf[pl.ds(i * NLANE, NLANE)] = rowid * NSUB + _bc_i32(goff)
                return c2
            lax.fori_loop(0, CH // NLANE, idx_body, 0)
            pltpu.async_copy(feat.at[ibuf], cbuf, sem)

        def process(cidx, ibuf, cbuf, sem):
            pltpu.make_async_copy(feat.at[ibuf], cbuf, sem).wait()
            l0 = cidx * CH
            rows_here = jnp.minimum(CH, lb - l0)

            def sub_body(sub, c1):
                t = tbuf[sub]
                basel = _bc_i32(sub * CAP * 16) + lanes
                dumpl = _bc_i32(DUMP) + lanes
                sixteen = _bc_i32(16)
                zero = _bc_i32(0)

                def oct_body(q, cv):
                    r = q * 8
                    xs = [cbuf[r + j, pl.ds(sub * NLANE, NLANE)]
                          for j in range(8)]
                    ms = [x >= t for x in xs]
                    inc = [jnp.where(m, sixteen, zero) for m in ms]
                    offs = [cv]
                    for j in range(7):
                        offs.append(offs[-1] + inc[j])
                    for j in range(8):
                        plsc.store_scatter(cand, [basel + offs[j]], xs[j],
                                           mask=ms[j])
                    return offs[7] + inc[7]
                nq = rows_here // 8
                cv = lax.fori_loop(0, nq, oct_body, cnt[sub])

                def row_body(r, cv2):
                    x = cbuf[r, pl.ds(sub * NLANE, NLANE)]
                    msk = x >= t
                    plsc.store_scatter(cand, [basel + cv2], x, mask=msk)
                    return cv2 + jnp.where(msk, sixteen, zero)
                cnt[sub] = lax.fori_loop(nq * 8, rows_here, row_body, cv)
                return c1
            lax.fori_loop(0, NSUB, sub_body, 0)

        @pl.when(nch > 0)
        def _():
            issue(0, idx0, chk0, sem0)

            def pair_body(p, c0):
                c_even = p * 2
                c_odd = c_even + 1

                @pl.when(c_odd < nch)
                def _():
                    issue(c_odd, idx1, chk1, sem1)
                process(c_even, idx0, chk0, sem0)

                @pl.when(c_even + 2 < nch)
                def _():
                    issue(c_even + 2, idx0, chk0, sem0)

                @pl.when(c_odd < nch)
                def _():
                    process(c_odd, idx1, chk1, sem1)
                flush_all(K)
                return c0
            lax.fori_loop(0, (nch + 1) // 2, pair_body, 0)
        flush_all(0)

        ki = jnp.minimum(K, lb)
        kif = _bc_f32(ki.astype(jnp.float32))

        def out_body(sub, c0):
            acc = _bc_f32(0.0)
            for i in range(K):
                sel = _bc_i32(i) < _bc_i32(ki)
                acc = acc + jnp.where(sel, run[sub * K + i], _bc_f32(0.0))
            outst[pl.ds(sub * NLANE, NLANE)] = acc / kif
            return c0
        lax.fori_loop(0, NSUB, out_body, 0)
        pltpu.sync_copy(outst, out_hbm.at[pl.ds(b * D + (task % NSUB) * 128, 128)])
        return 0

    lax.fori_loop(0, NTASK, task_body, 0)


_sc_call = pl.kernel(
    _tec_body,
    out_type=jax.ShapeDtypeStruct((B * D,), jnp.float32),
    mesh=plsc.VectorSubcoreMesh(core_axis_name="c", subcore_axis_name="s"),
    compiler_params=pltpu.CompilerParams(needs_layout_passes=False),
    scratch_types=[
        pltpu.VMEM((16,), jnp.int32),
        pltpu.VMEM((CH,), jnp.int32),
        pltpu.VMEM((CH,), jnp.int32),
        pltpu.VMEM((CH, NSUB * 16), jnp.float32),
        pltpu.VMEM((CH, NSUB * 16), jnp.float32),
        pltpu.VMEM((NSUB * CAP * 16 + 16,), jnp.float32),
        pltpu.VMEM((NSUB * K, 16), jnp.float32),
        pltpu.VMEM((NSUB, 16), jnp.int32),
        pltpu.VMEM((NSUB, 16), jnp.float32),
        pltpu.VMEM((128,), jnp.float32),
        pltpu.SemaphoreType.DMA,
        pltpu.SemaphoreType.DMA,
    ],
)


@jax.jit
def kernel(features, lengths):
    feat2 = features.reshape(B * L * NSUB, 128)
    return _sc_call(feat2, lengths).reshape(B, D)


# speedup vs baseline: 1.1076x; 1.0691x over previous
import jax
import jax.numpy as jnp
from jax import lax
from jax.experimental import pallas as pl
from jax.experimental.pallas import tpu as pltpu
from jax.experimental.pallas import tpu_sc as plsc

B, L, D, K = 16, 4096, 1024, 32
CH = 128
CAP = 384
NSUB = 8
NTASK = 4
NLANE = 16
DUMP = 8 * 384 * 16

NEG = float("-inf")


def _bc_i32(s):
    return jnp.full((NLANE,), s, dtype=jnp.int32)


def _bc_f32(s):
    return jnp.full((NLANE,), s, dtype=jnp.float32)


def _ce_desc(a, b):
    return jnp.maximum(a, b), jnp.minimum(a, b)


def _bitonic_sort_desc(a):
    n = len(a)
    k = 2
    while k <= n:
        j = k // 2
        while j >= 1:
            for i in range(n):
                l = i ^ j
                if l > i:
                    hi, lo = _ce_desc(a[i], a[l])
                    if (i & k) == 0:
                        a[i], a[l] = hi, lo
                    else:
                        a[i], a[l] = lo, hi
            j //= 2
        k *= 2
    return a


def _bitonic_cleanup_desc(a):
    n = len(a)
    j = n // 2
    while j >= 1:
        for i in range(n):
            l = i ^ j
            if l > i:
                a[i], a[l] = _ce_desc(a[i], a[l])
        j //= 2
    return a


def _smax_i32(v, lanes):
    for d in (8, 4, 2, 1):
        perm = jnp.bitwise_xor(lanes, _bc_i32(d))
        v = jnp.maximum(v, v.at[perm].get(mode="promise_in_bounds"))
    return v[0]


def _tec_body(feat, len_hbm, out_hbm, len_v, idx0, idx1, chk0, chk1, cand, run,
              cnt, tbuf, outst, sem0, sem1):
    wid = lax.axis_index("s") * 2 + lax.axis_index("c")
    pltpu.sync_copy(len_hbm, len_v)
    lanes = lax.iota(jnp.int32, 16)

    def task_body(ti, _):
        task = ti * 32 + wid
        b = task // NSUB
        g0 = (task % NSUB) * NSUB
        lb = len_v[...].at[_bc_i32(b)].get(mode="promise_in_bounds")[0]

        def reset_run(i, c):
            run[i] = _bc_f32(NEG)
            return c
        lax.fori_loop(0, NSUB * K, reset_run, 0)

        def reset_state(i, c):
            cnt[i] = _bc_i32(0)
            tbuf[i] = _bc_f32(NEG)
            return c
        lax.fori_loop(0, NSUB, reset_state, 0)

        def flush_all(thresh):
            def per_sub(sub, c0):
                c = cnt[sub]
                maxc = _smax_i32(c, lanes)

                @pl.when(maxc > thresh * 16)
                def _():
                    nblk = (maxc + K * 16 - 1) // (K * 16)

                    def blk_body(blk, c1):
                        base = (sub * CAP + blk * K) * 16
                        s = []
                        for i in range(K):
                            x = cand[pl.ds(base + i * 16, 16)]
                            valid = _bc_i32((blk * K + i) * 16) < c
                            s.append(jnp.where(valid, x, _bc_f32(NEG)))
                        s = _bitonic_sort_desc(s)
                        rbase = sub * K
                        m = [jnp.maximum(run[rbase + i], s[K - 1 - i])
                             for i in range(K)]
                        m = _bitonic_cleanup_desc(m)
                        for i in range(K):
                            run[rbase + i] = m[i]
                        return c1
                    lax.fori_loop(0, nblk, blk_body, 0)
                    cnt[sub] = _bc_i32(0)
                    tbuf[sub] = run[sub * K + K - 1]
                return c0
            lax.fori_loop(0, NSUB, per_sub, 0)

        nch = (lb + CH - 1) // CH

        goff = task % NSUB

        def issue(cidx, ibuf, cbuf, sem):
            l0 = cidx * CH

            def idx_body(i, c2):
                rowid = _bc_i32(b * L + l0 + i * NLANE) + lanes
                ibuf[pl.ds(i * NLANE, NLANE)] = rowid * NSUB + _bc_i32(goff)
                return c2
            lax.fori_loop(0, CH // NLANE, idx_body, 0)
            pltpu.async_copy(feat.at[ibuf], cbuf, sem)

        def process(cidx, ibuf, cbuf, sem):
            pltpu.make_async_copy(feat.at[ibuf], cbuf, sem).wait()
            l0 = cidx * CH
            rows_here = jnp.minimum(CH, lb - l0)

            def sub_body(sub, c1):
                t = tbuf[sub]
                basel = _bc_i32(sub * CAP * 16) + lanes
                sixteen = _bc_i32(16)
                zero = _bc_i32(0)

                def blk_body16(q, cv):
                    r = q * 16
                    xs = [cbuf[r + j, pl.ds(sub * NLANE, NLANE)]
                          for j in range(16)]
                    ms = [x >= t for x in xs]
                    inc = [jnp.where(m, sixteen, zero) for m in ms]
                    offs = [cv]
                    for j in range(15):
                        offs.append(offs[-1] + inc[j])
                    for j in range(16):
                        plsc.store_scatter(cand, [offs[j]], xs[j], mask=ms[j])
                    return offs[15] + inc[15]
                nq = rows_here // 16
                cv = lax.fori_loop(0, nq, blk_body16, cnt[sub] + basel)

                def row_body(r, cv2):
                    x = cbuf[r, pl.ds(sub * NLANE, NLANE)]
                    msk = x >= t
                    plsc.store_scatter(cand, [cv2], x, mask=msk)
                    return cv2 + jnp.where(msk, sixteen, zero)
                cv = lax.fori_loop(nq * 16, rows_here, row_body, cv)
                cnt[sub] = cv - basel
                return c1
            lax.fori_loop(0, NSUB, sub_body, 0)

        @pl.when(nch > 0)
        def _():
            issue(0, idx0, chk0, sem0)

            def pair_body(p, c0):
                c_even = p * 2
                c_odd = c_even + 1

                @pl.when(c_odd < nch)
                def _():
                    issue(c_odd, idx1, chk1, sem1)
                process(c_even, idx0, chk0, sem0)

                @pl.when(c_even + 2 < nch)
                def _():
                    issue(c_even + 2, idx0, chk0, sem0)

                @pl.when(c_odd < nch)
                def _():
                    process(c_odd, idx1, chk1, sem1)
                flush_all(K)
                return c0
            lax.fori_loop(0, (nch + 1) // 2, pair_body, 0)
        flush_all(0)

        ki = jnp.minimum(K, lb)
        kif = _bc_f32(ki.astype(jnp.float32))

        def out_body(sub, c0):
            acc = _bc_f32(0.0)
            for i in range(K):
                sel = _bc_i32(i) < _bc_i32(ki)
                acc = acc + jnp.where(sel, run[sub * K + i], _bc_f32(0.0))
            outst[pl.ds(sub * NLANE, NLANE)] = acc / kif
            return c0
        lax.fori_loop(0, NSUB, out_body, 0)
        pltpu.sync_copy(outst, out_hbm.at[pl.ds(b * D + (task % NSUB) * 128, 128)])
        return 0

    lax.fori_loop(0, NTASK, task_body, 0)


_sc_call = pl.kernel(
    _tec_body,
    out_type=jax.ShapeDtypeStruct((B * D,), jnp.float32),
    mesh=plsc.VectorSubcoreMesh(core_axis_name="c", subcore_axis_name="s"),
    compiler_params=pltpu.CompilerParams(needs_layout_passes=False),
    scratch_types=[
        pltpu.VMEM((16,), jnp.int32),
        pltpu.VMEM((CH,), jnp.int32),
        pltpu.VMEM((CH,), jnp.int32),
        pltpu.VMEM((CH, NSUB * 16), jnp.float32),
        pltpu.VMEM((CH, NSUB * 16), jnp.float32),
        pltpu.VMEM((NSUB * CAP * 16 + 16,), jnp.float32),
        pltpu.VMEM((NSUB * K, 16), jnp.float32),
        pltpu.VMEM((NSUB, 16), jnp.int32),
        pltpu.VMEM((NSUB, 16), jnp.float32),
        pltpu.VMEM((128,), jnp.float32),
        pltpu.SemaphoreType.DMA,
        pltpu.SemaphoreType.DMA,
    ],
)


@jax.jit
def kernel(features, lengths):
    feat2 = features.reshape(B * L * NSUB, 128)
    return _sc_call(feat2, lengths).reshape(B, D)


# speedup vs baseline: 1.1081x; 1.0005x over previous
import jax
import jax.numpy as jnp
from jax import lax
from jax.experimental import pallas as pl
from jax.experimental.pallas import tpu as pltpu
from jax.experimental.pallas import tpu_sc as plsc

B, L, D, K = 16, 4096, 1024, 32
CH = 128
CAP = 384
NSUB = 8
NTASK = 4
NLANE = 16

NEG = float("-inf")


def _bc_i32(s):
    return jnp.full((NLANE,), s, dtype=jnp.int32)


def _bc_f32(s):
    return jnp.full((NLANE,), s, dtype=jnp.float32)


def _ce_desc(a, b):
    return jnp.maximum(a, b), jnp.minimum(a, b)


def _bitonic_sort_desc(a):
    n = len(a)
    k = 2
    while k <= n:
        j = k // 2
        while j >= 1:
            for i in range(n):
                l = i ^ j
                if l > i:
                    hi, lo = _ce_desc(a[i], a[l])
                    if (i & k) == 0:
                        a[i], a[l] = hi, lo
                    else:
                        a[i], a[l] = lo, hi
            j //= 2
        k *= 2
    return a


def _bitonic_cleanup_desc(a):
    n = len(a)
    j = n // 2
    while j >= 1:
        for i in range(n):
            l = i ^ j
            if l > i:
                a[i], a[l] = _ce_desc(a[i], a[l])
        j //= 2
    return a


def _smax_i32(v, lanes):
    for d in (8, 4, 2, 1):
        perm = jnp.bitwise_xor(lanes, _bc_i32(d))
        v = jnp.maximum(v, v.at[perm].get(mode="promise_in_bounds"))
    return v[0]


def _tec_body(feat, len_hbm, out_hbm, len_v, idx0, idx1, chk0, chk1, cand, run,
              cnt, tbuf, outst, sem0, sem1):
    wid = lax.axis_index("s") * 2 + lax.axis_index("c")
    pltpu.sync_copy(len_hbm, len_v)
    lanes = lax.iota(jnp.int32, 16)

    def task_body(ti, _):
        task = ti * 32 + wid
        b = task // NSUB
        g0 = (task % NSUB) * NSUB
        lb = len_v[...].at[_bc_i32(b)].get(mode="promise_in_bounds")[0]

        def reset_run(i, c):
            run[i] = _bc_f32(NEG)
            return c
        lax.fori_loop(0, NSUB * K, reset_run, 0)

        def reset_state(i, c):
            cnt[i] = _bc_i32(0)
            tbuf[i] = _bc_f32(NEG)
            return c
        lax.fori_loop(0, NSUB, reset_state, 0)

        def flush_all(thresh):
            def per_sub(sub, c0):
                c = cnt[sub]
                maxc = _smax_i32(c, lanes)

                @pl.when(maxc > thresh * 16)
                def _():
                    nblk = (maxc + K * 16 - 1) // (K * 16)

                    def blk_body(blk, c1):
                        base = (sub * CAP + blk * K) * 16
                        s = []
                        for i in range(K):
                            x = cand[pl.ds(base + i * 16, 16)]
                            valid = _bc_i32((blk * K + i) * 16) < c
                            s.append(jnp.where(valid, x, _bc_f32(NEG)))
                        s = _bitonic_sort_desc(s)
                        rbase = sub * K
                        m = [jnp.maximum(run[rbase + i], s[K - 1 - i])
                             for i in range(K)]
                        m = _bitonic_cleanup_desc(m)
                        for i in range(K):
                            run[rbase + i] = m[i]
                        return c1
                    lax.fori_loop(0, nblk, blk_body, 0)
                    cnt[sub] = _bc_i32(0)
                    tbuf[sub] = run[sub * K + K - 1]
                return c0
            lax.fori_loop(0, NSUB, per_sub, 0)

        nch = (lb + CH - 1) // CH

        goff = task % NSUB

        def issue(cidx, ibuf, cbuf, sem):
            l0 = cidx * CH

            def idx_body(i, c2):
                rowid = _bc_i32(b * L + l0 + i * NLANE) + lanes
                ibuf[pl.ds(i * NLANE, NLANE)] = rowid * NSUB + _bc_i32(goff)
                return c2
            lax.fori_loop(0, CH // NLANE, idx_body, 0)
            pltpu.async_copy(feat.at[ibuf], cbuf, sem)

        def process(cidx, ibuf, cbuf, sem):
            pltpu.make_async_copy(feat.at[ibuf], cbuf, sem).wait()
            l0 = cidx * CH
            rows_here = jnp.minimum(CH, lb - l0)

            def sub_body(sub, c1):
                t = tbuf[sub]
                basel = _bc_i32(sub * CAP * 16) + lanes
                sixteen = _bc_i32(16)
                zero = _bc_i32(0)

                def blk_body16(q, cv):
                    r = q * 16
                    xs = [cbuf[r + j, pl.ds(sub * NLANE, NLANE)]
                          for j in range(16)]
                    ms = [x >= t for x in xs]
                    inc = [jnp.where(m, sixteen, zero) for m in ms]
                    offs = [cv]
                    for j in range(15):
                        offs.append(offs[-1] + inc[j])
                    for j in range(16):
                        plsc.store_scatter(cand, [offs[j]], xs[j], mask=ms[j])
                    return offs[15] + inc[15]
                nq = rows_here // 16
                cv = lax.fori_loop(0, nq, blk_body16, cnt[sub] + basel)

                def row_body(r, cv2):
                    x = cbuf[r, pl.ds(sub * NLANE, NLANE)]
                    msk = x >= t
                    plsc.store_scatter(cand, [cv2], x, mask=msk)
                    return cv2 + jnp.where(msk, sixteen, zero)
                cv = lax.fori_loop(nq * 16, rows_here, row_body, cv)
                cnt[sub] = cv - basel
                return c1
            lax.fori_loop(0, NSUB, sub_body, 0)

        @pl.when(nch > 0)
        def _():
            issue(0, idx0, chk0, sem0)

            def pair_body(p, c0):
                c_even = p * 2
                c_odd = c_even + 1

                @pl.when(c_odd < nch)
                def _():
                    issue(c_odd, idx1, chk1, sem1)
                process(c_even, idx0, chk0, sem0)

                @pl.when(c_even + 2 < nch)
                def _():
                    issue(c_even + 2, idx0, chk0, sem0)

                @pl.when(c_odd < nch)
                def _():
                    process(c_odd, idx1, chk1, sem1)
                flush_all(K)
                return c0
            lax.fori_loop(0, (nch + 1) // 2, pair_body, 0)
        flush_all(0)

        ki = jnp.minimum(K, lb)
        kif = _bc_f32(ki.astype(jnp.float32))

        def out_body(sub, c0):
            acc = _bc_f32(0.0)
            for i in range(K):
                sel = _bc_i32(i) < _bc_i32(ki)
                acc = acc + jnp.where(sel, run[sub * K + i], _bc_f32(0.0))
            outst[pl.ds(sub * NLANE, NLANE)] = acc / kif
            return c0
        lax.fori_loop(0, NSUB, out_body, 0)
        pltpu.sync_copy(outst, out_hbm.at[pl.ds(b * D + (task % NSUB) * 128, 128)])
        return 0

    lax.fori_loop(0, NTASK, task_body, 0)


_sc_call = pl.kernel(
    _tec_body,
    out_type=jax.ShapeDtypeStruct((B * D,), jnp.float32),
    mesh=plsc.VectorSubcoreMesh(core_axis_name="c", subcore_axis_name="s"),
    compiler_params=pltpu.CompilerParams(needs_layout_passes=False),
    scratch_types=[
        pltpu.VMEM((16,), jnp.int32),
        pltpu.VMEM((CH,), jnp.int32),
        pltpu.VMEM((CH,), jnp.int32),
        pltpu.VMEM((CH, NSUB * 16), jnp.float32),
        pltpu.VMEM((CH, NSUB * 16), jnp.float32),
        pltpu.VMEM((NSUB * CAP * 16,), jnp.float32),
        pltpu.VMEM((NSUB * K, 16), jnp.float32),
        pltpu.VMEM((NSUB, 16), jnp.int32),
        pltpu.VMEM((NSUB, 16), jnp.float32),
        pltpu.VMEM((128,), jnp.float32),
        pltpu.SemaphoreType.DMA,
        pltpu.SemaphoreType.DMA,
    ],
)


@jax.jit
def kernel(features, lengths):
    feat2 = features.reshape(B * L * NSUB, 128)
    return _sc_call(feat2, lengths).reshape(B, D)


# speedup vs baseline: 1.1111x; 1.0028x over previous
import jax
import jax.numpy as jnp
from jax import lax
from jax.experimental import pallas as pl
from jax.experimental.pallas import tpu as pltpu
from jax.experimental.pallas import tpu_sc as plsc

B, L, D, K = 16, 4096, 1024, 32
CH = 128
CAP = 384
NSUB = 8
NTASK = 4
NLANE = 16

NEG = float("-inf")


def _bc_i32(s):
    return jnp.full((NLANE,), s, dtype=jnp.int32)


def _bc_f32(s):
    return jnp.full((NLANE,), s, dtype=jnp.float32)


def _ce_desc(a, b):
    return jnp.maximum(a, b), jnp.minimum(a, b)


def _bitonic_sort_desc(a):
    n = len(a)
    k = 2
    while k <= n:
        j = k // 2
        while j >= 1:
            for i in range(n):
                l = i ^ j
                if l > i:
                    hi, lo = _ce_desc(a[i], a[l])
                    if (i & k) == 0:
                        a[i], a[l] = hi, lo
                    else:
                        a[i], a[l] = lo, hi
            j //= 2
        k *= 2
    return a


def _bitonic_cleanup_desc(a):
    n = len(a)
    j = n // 2
    while j >= 1:
        for i in range(n):
            l = i ^ j
            if l > i:
                a[i], a[l] = _ce_desc(a[i], a[l])
        j //= 2
    return a


def _smax_i32(v, lanes):
    for d in (8, 4, 2, 1):
        perm = jnp.bitwise_xor(lanes, _bc_i32(d))
        v = jnp.maximum(v, v.at[perm].get(mode="promise_in_bounds"))
    return v[0]


def _tec_body(feat, len_hbm, out_hbm, len_v, idx0, idx1, chk0, chk1, cand, run,
              cnt, tbuf, outst, sem0, sem1):
    wid = lax.axis_index("s") * 2 + lax.axis_index("c")
    pltpu.sync_copy(len_hbm, len_v)
    lanes = lax.iota(jnp.int32, 16)

    def task_body(ti, _):
        task = ti * 32 + wid
        b = task // NSUB
        g0 = (task % NSUB) * NSUB
        lb = len_v[...].at[_bc_i32(b)].get(mode="promise_in_bounds")[0]

        def reset_run(i, c):
            run[i] = _bc_f32(NEG)
            return c
        lax.fori_loop(0, NSUB * K, reset_run, 0)

        def reset_state(i, c):
            cnt[i] = _bc_i32(0)
            tbuf[i] = _bc_f32(NEG)
            return c
        lax.fori_loop(0, NSUB, reset_state, 0)

        def flush_all(thresh):
            def per_sub(sub, c0):
                c = cnt[sub]
                maxc = _smax_i32(c, lanes)

                @pl.when(maxc > thresh * 16)
                def _():
                    nblk = (maxc + K * 16 - 1) // (K * 16)

                    def blk_body(blk, c1):
                        base = (sub * CAP + blk * K) * 16
                        s = []
                        for i in range(K):
                            x = cand[pl.ds(base + i * 16, 16)]
                            valid = _bc_i32((blk * K + i) * 16) < c
                            s.append(jnp.where(valid, x, _bc_f32(NEG)))
                        s = _bitonic_sort_desc(s)
                        rbase = sub * K
                        m = [jnp.maximum(run[rbase + i], s[K - 1 - i])
                             for i in range(K)]
                        m = _bitonic_cleanup_desc(m)
                        for i in range(K):
                            run[rbase + i] = m[i]
                        return c1
                    lax.fori_loop(0, nblk, blk_body, 0)
                    cnt[sub] = _bc_i32(0)
                    tbuf[sub] = run[sub * K + K - 1]
                return c0
            lax.fori_loop(0, NSUB, per_sub, 0)

        nch = (lb + CH - 1) // CH

        goff = task % NSUB

        def issue(cidx, ibuf, cbuf, sem):
            l0 = cidx * CH

            def idx_body(i, c2):
                rowid = _bc_i32(b * L + l0 + i * NLANE) + lanes
                ibuf[pl.ds(i * NLANE, NLANE)] = rowid * NSUB + _bc_i32(goff)
                return c2
            lax.fori_loop(0, CH // NLANE, idx_body, 0)
            pltpu.async_copy(feat.at[ibuf], cbuf, sem)

        def process(cidx, ibuf, cbuf, sem):
            pltpu.make_async_copy(feat.at[ibuf], cbuf, sem).wait()
            l0 = cidx * CH
            rows_here = jnp.minimum(CH, lb - l0)

            def sub_body(sub, c1):
                t = tbuf[sub]
                basel = _bc_i32(sub * CAP * 16) + lanes
                sixteen = _bc_i32(16)
                zero = _bc_i32(0)

                def blk_body16(q, cv):
                    r = q * 16
                    xs = [cbuf[r + j, pl.ds(sub * NLANE, NLANE)]
                          for j in range(16)]
                    ms = [x >= t for x in xs]
                    inc = [jnp.where(m, sixteen, zero) for m in ms]
                    offs = [cv]
                    for j in range(15):
                        offs.append(offs[-1] + inc[j])
                    for j in range(16):
                        plsc.store_scatter(cand, [offs[j]], xs[j], mask=ms[j])
                    return offs[15] + inc[15]
                nq = rows_here // 16
                cv = lax.fori_loop(0, nq, blk_body16, cnt[sub] + basel)

                def row_body(r, cv2):
                    x = cbuf[r, pl.ds(sub * NLANE, NLANE)]
                    msk = x >= t
                    plsc.store_scatter(cand, [cv2], x, mask=msk)
                    return cv2 + jnp.where(msk, sixteen, zero)
                cv = lax.fori_loop(nq * 16, rows_here, row_body, cv)
                cnt[sub] = cv - basel
                return c1
            lax.fori_loop(0, NSUB, sub_body, 0)

        @pl.when(nch > 0)
        def _():
            issue(0, idx0, chk0, sem0)

            def pair_body(p, c0):
                c_even = p * 2
                c_odd = c_even + 1

                @pl.when(c_odd < nch)
                def _():
                    issue(c_odd, idx1, chk1, sem1)
                process(c_even, idx0, chk0, sem0)

                @pl.when(c_even + 2 < nch)
                def _():
                    issue(c_even + 2, idx0, chk0, sem0)

                @pl.when(c_odd < nch)
                def _():
                    process(c_odd, idx1, chk1, sem1)
                flush_all(K)
                return c0
            lax.fori_loop(0, (nch + 1) // 2, pair_body, 0)
        flush_all(0)

        ki = jnp.minimum(K, lb)
        kif = _bc_f32(ki.astype(jnp.float32))

        def out_body(sub, c0):
            acc = _bc_f32(0.0)
            for i in range(K):
                sel = _bc_i32(i) < _bc_i32(ki)
                acc = acc + jnp.where(sel, run[sub * K + i], _bc_f32(0.0))
            outst[pl.ds(sub * NLANE, NLANE)] = acc / kif
            return c0
        lax.fori_loop(0, NSUB, out_body, 0)
        pltpu.sync_copy(outst, out_hbm.at[pl.ds(b * D + (task % NSUB) * 128, 128)])
        return 0

    lax.fori_loop(0, NTASK, task_body, 0)


import functools


@functools.cache
def _make_sc_call():
    return pl.kernel(
    _tec_body,
    out_type=jax.ShapeDtypeStruct((B * D,), jnp.float32),
    mesh=plsc.VectorSubcoreMesh(core_axis_name="c", subcore_axis_name="s",
                                num_cores=2, num_subcores=16),
    compiler_params=pltpu.CompilerParams(needs_layout_passes=False),
    scratch_types=[
        pltpu.VMEM((16,), jnp.int32),
        pltpu.VMEM((CH,), jnp.int32),
        pltpu.VMEM((CH,), jnp.int32),
        pltpu.VMEM((CH, NSUB * 16), jnp.float32),
        pltpu.VMEM((CH, NSUB * 16), jnp.float32),
        pltpu.VMEM((NSUB * CAP * 16,), jnp.float32),
        pltpu.VMEM((NSUB * K, 16), jnp.float32),
        pltpu.VMEM((NSUB, 16), jnp.int32),
        pltpu.VMEM((NSUB, 16), jnp.float32),
        pltpu.VMEM((128,), jnp.float32),
        pltpu.SemaphoreType.DMA,
        pltpu.SemaphoreType.DMA,
    ],
    )


@jax.jit
def kernel(features, lengths):
    feat2 = features.reshape(B * L * NSUB, 128)
    return _make_sc_call()(feat2, lengths).reshape(B, D)
